# baseline (speedup 1.0000x reference)
import functools

import jax
import jax.numpy as jnp
from jax import lax
from jax.experimental import pallas as pl
from jax.experimental.pallas import tpu as pltpu
from jax.experimental.pallas import tpu_sc as plsc

T = 8
NN = 10000
E = 512000
F0 = 128
F1 = 32
F2 = 32
HALF = 16
TN = T * NN

NB = 10
NBLK = NN // NB
EROWS = E // 128
ERP = 4096
ER_B = EROWS // NB

NC = 2
NS = 16
CH = 128

_PREC = lax.Precision.HIGHEST

_GDN = lax.GatherDimensionNumbers(
    offset_dims=(), collapsed_slice_dims=(0,), start_index_map=(0,))


def _bcast_lane(vec, lane):
    idx = jnp.full((16, 1), lane, jnp.int32)
    return lax.gather(vec, idx, _GDN, (1,),
                      mode=lax.GatherScatterMode.PROMISE_IN_BOUNDS)


def _stage1_body(m_ref, x_ref, w_ref, et_ref, es_ref, etr_ref,
                 p0_ref, p1_ref, sft_ref):
    w = w_ref[...]
    for t in range(T):
        acc = None
        for u in range(T):
            wt = m_ref[t, u] * w[t]
            term = lax.dot_general(
                x_ref[u], wt, (((1,), (0,)), ((), ())),
                precision=_PREC, preferred_element_type=jnp.float32)
            acc = term if acc is None else acc + term
        p0_ref[t] = acc[:, :HALF]
        p1_ref[t] = acc[:, HALF:]
    tm = et_ref[...] * NN
    sft_ref[0] = tm + es_ref[...]
    sft_ref[1] = tm + etr_ref[...]


def _run_stage1(M, X, W, et3, es3, etr3):
    return pl.pallas_call(
        _stage1_body,
        grid=(NB,),
        in_specs=[
            pl.BlockSpec(memory_space=pltpu.SMEM),
            pl.BlockSpec((T, NBLK, F0), lambda i: (0, i, 0)),
            pl.BlockSpec((T, F0, F1), lambda i: (0, 0, 0)),
            pl.BlockSpec((1, ER_B, 128), lambda i: (i, 0, 0)),
            pl.BlockSpec((1, ER_B, 128), lambda i: (i, 0, 0)),
            pl.BlockSpec((1, ER_B, 128), lambda i: (i, 0, 0)),
        ],
        out_specs=[
            pl.BlockSpec((T, NBLK, HALF), lambda i: (0, i, 0)),
            pl.BlockSpec((T, NBLK, HALF), lambda i: (0, i, 0)),
            pl.BlockSpec((2, 1, ER_B, 128), lambda i: (0, i, 0, 0)),
        ],
        out_shape=[
            jax.ShapeDtypeStruct((T, NN, HALF), jnp.float32),
            jax.ShapeDtypeStruct((T, NN, HALF), jnp.float32),
            jax.ShapeDtypeStruct((2, NB, ER_B, 128), jnp.int32),
        ],
    )(M, X, W, et3, es3, etr3)


SB = 16
NSB = ERP // NS // SB
ZROWS = TN // NS


def _segsum_body(p0_hbm, p1_hbm, a_hbm, sft_hbm, z_hbm,
                 s0_hbm, s1_hbm,
                 acc, sft_blk, a_blk, rows, sbuf,
                 sg0, sg1, sg2, sg3, sa0, sa1, sa2, sa3):
    semg = (sg0, sg1, sg2, sg3)
    sema = (sa0, sa1, sa2, sa3)
    c = lax.axis_index("c")
    s = lax.axis_index("s")
    pltpu.sync_copy(z_hbm, acc.at[pl.ds(s * ZROWS, ZROWS)])
    plsc.subcore_barrier()

    base_row = s * (ERP // NS)

    def fire_gather(slot, k, r):
        @pl.when(c == 0)
        def _():
            pltpu.make_async_copy(
                p0_hbm.at[sft_blk.at[slot, 1, r]], rows.at[k], semg[k]).start()

        @pl.when(c == 1)
        def _():
            pltpu.make_async_copy(
                p1_hbm.at[sft_blk.at[slot, 1, r]], rows.at[k], semg[k]).start()

    def wait_gather(k):
        pltpu.make_async_copy(
            p0_hbm.at[pl.ds(0, CH)], rows.at[k], semg[k]).wait()

    def drain_scatter(k):
        pltpu.make_async_copy(
            p0_hbm.at[pl.ds(0, CH)], sbuf.at[k], sema[k]).wait()

    def scale(slot, k, r):
        def grp(g, carry):
            a_vec = a_blk[slot, r, pl.ds(g * 16, 16)]
            base = g * 16
            for ee in range(16):
                bc = _bcast_lane(a_vec, ee)
                sbuf[k, base + ee] = rows[k, base + ee] * bc
            return carry

        lax.fori_loop(0, CH // 16, grp, 0)

    def fire_scatter(slot, k, r):
        pltpu.make_async_copy(
            sbuf.at[k], acc.at[sft_blk.at[slot, 0, r]], sema[k]).start(add=True)

    def run_superblock(sb_idx, slot, first_pred):
        r0 = base_row + sb_idx * SB
        pltpu.sync_copy(sft_hbm.at[:, pl.ds(r0, SB)], sft_blk.at[slot])
        pltpu.sync_copy(a_hbm.at[pl.ds(r0, SB)], a_blk.at[slot])
        for k in range(4):
            fire_gather(slot, k, k)

        def chunk(u, r, drain):
            wait_gather(u)
            if drain == "always":
                drain_scatter(u)
            elif drain == "cond":
                @pl.when(jnp.logical_not(first_pred))
                def _():
                    drain_scatter(u)
            scale(slot, u, r)
            fire_scatter(slot, u, r)

        for u in range(4):
            chunk(u, u, "cond" if first_pred is not None else "always")
            fire_gather(slot, u, u + 4)

        def quad(q, carry):
            for u in range(4):
                r = q * 4 + u
                chunk(u, r, "always")
                fire_gather(slot, u, r + 4)
            return carry

        lax.fori_loop(1, SB // 4 - 1, quad, 0)
        for u in range(4):
            chunk(u, SB - 4 + u, "always")

    def sbpair(p, carry):
        run_superblock(p * 2, 0, p == 0)
        run_superblock(p * 2 + 1, 1, None)
        return carry

    lax.fori_loop(0, NSB // 2, sbpair, 0)
    for k in range(4):
        drain_scatter(k)
    plsc.subcore_barrier()

    wr0 = s * ZROWS

    @pl.when(c == 0)
    def _():
        pltpu.sync_copy(acc.at[pl.ds(wr0, ZROWS)], s0_hbm.at[pl.ds(wr0, ZROWS)])

    @pl.when(c == 1)
    def _():
        pltpu.sync_copy(acc.at[pl.ds(wr0, ZROWS)], s1_hbm.at[pl.ds(wr0, ZROWS)])


_segsum = functools.partial(
    pl.kernel,
    out_type=[jax.ShapeDtypeStruct((TN, HALF), jnp.float32),
              jax.ShapeDtypeStruct((TN, HALF), jnp.float32)],
    mesh=plsc.VectorSubcoreMesh(core_axis_name="c", subcore_axis_name="s"),
    scratch_types=[
        pltpu.VMEM_SHARED((TN, HALF), jnp.float32),
        pltpu.VMEM((2, 2, SB, CH), jnp.int32),
        pltpu.VMEM((2, SB, CH), jnp.float32),
        pltpu.VMEM((4, CH, HALF), jnp.float32),
        pltpu.VMEM((4, CH, HALF), jnp.float32),
        pltpu.SemaphoreType.DMA, pltpu.SemaphoreType.DMA,
        pltpu.SemaphoreType.DMA, pltpu.SemaphoreType.DMA,
        pltpu.SemaphoreType.DMA, pltpu.SemaphoreType.DMA,
        pltpu.SemaphoreType.DMA, pltpu.SemaphoreType.DMA,
    ],
    compiler_params=pltpu.CompilerParams(use_tc_tiling_on_sc=False),
)(_segsum_body)


def _proj_body(minv_ref, u_ref, s0_ref, s1_ref, zs_ref, zt_ref):
    u = u_ref[...]
    u0s, u1s = u[0:HALF], u[HALF:2 * HALF]
    u0t, u1t = u[2 * HALF:3 * HALF], u[3 * HALF:]
    for t in range(T):
        y0 = minv_ref[t, 0] * s0_ref[0]
        y1 = minv_ref[t, 0] * s1_ref[0]
        for uu in range(1, T):
            y0 = y0 + minv_ref[t, uu] * s0_ref[uu]
            y1 = y1 + minv_ref[t, uu] * s1_ref[uu]
        zs_ref[t] = (
            lax.dot_general(y0, u0s, (((1,), (0,)), ((), ())),
                            precision=_PREC, preferred_element_type=jnp.float32)
            + lax.dot_general(y1, u1s, (((1,), (0,)), ((), ())),
                              precision=_PREC, preferred_element_type=jnp.float32))
        zt_ref[t] = (
            lax.dot_general(y0, u0t, (((1,), (0,)), ((), ())),
                            precision=_PREC, preferred_element_type=jnp.float32)
            + lax.dot_general(y1, u1t, (((1,), (0,)), ((), ())),
                              precision=_PREC, preferred_element_type=jnp.float32))


def _run_proj(Minv, U, S0, S1):
    return pl.pallas_call(
        _proj_body,
        grid=(NB,),
        in_specs=[
            pl.BlockSpec(memory_space=pltpu.SMEM),
            pl.BlockSpec((2 * F1, F2), lambda i: (0, 0)),
            pl.BlockSpec((T, NBLK, HALF), lambda i: (0, i, 0)),
            pl.BlockSpec((T, NBLK, HALF), lambda i: (0, i, 0)),
        ],
        out_specs=[
            pl.BlockSpec((T, NBLK, F2), lambda i: (0, i, 0)),
            pl.BlockSpec((T, NBLK, F2), lambda i: (0, i, 0)),
        ],
        out_shape=[
            jax.ShapeDtypeStruct((T, NN, F2), jnp.float32),
            jax.ShapeDtypeStruct((T, NN, F2), jnp.float32),
        ],
    )(Minv, U, S0, S1)


NW = NC * NS
PAGES = EROWS // 8


def _edgeout_body(zs_hbm, zt_hbm, sft_hbm, out_hbm,
                  sftw, bs, bt, bw,
                  sg0, sg1, sg2, sg3, sw0, sw1, sw2, sw3):
    semg = (sg0, sg1, sg2, sg3)
    semw = (sw0, sw1, sw2, sw3)
    c = lax.axis_index("c")
    s = lax.axis_index("s")
    wid = s * NC + c
    extra = PAGES % NW
    npages = jnp.int32(PAGES // NW) + (wid < extra).astype(jnp.int32)
    page0 = jnp.where(wid < extra, wid * (PAGES // NW + 1),
                      extra + wid * (PAGES // NW))
    row0 = page0 * 8
    nquads = npages * 2

    pltpu.sync_copy(sft_hbm.at[:, pl.ds(row0, 128)], sftw)

    def fire_gathers(k, m):
        pltpu.make_async_copy(
            zs_hbm.at[sftw.at[0, m]], bs.at[k], semg[k]).start()
        pltpu.make_async_copy(
            zt_hbm.at[sftw.at[1, m]], bt.at[k], semg[k]).start()

    def wait_gathers(k):
        pltpu.make_async_copy(
            zs_hbm.at[pl.ds(0, CH)], bs.at[k], semg[k]).wait()
        pltpu.make_async_copy(
            zs_hbm.at[pl.ds(0, CH)], bt.at[k], semg[k]).wait()

    def drain_write(k):
        pltpu.make_async_copy(
            zs_hbm.at[pl.ds(0, CH)], bw.at[k], semw[k]).wait()

    for k in range(4):
        fire_gathers(k, k)

    def quad(q, carry):
        for u in range(4):
            m = q * 4 + u
            wait_gathers(u)

            @pl.when(q > 0)
            def _():
                drain_write(u)

            def row_add(r, carry2):
                bw[u, r, pl.ds(0, 16)] = (bs[u, r, pl.ds(0, 16)]
                                          + bt[u, r, pl.ds(0, 16)])
                bw[u, r, pl.ds(16, 16)] = (bs[u, r, pl.ds(16, 16)]
                                           + bt[u, r, pl.ds(16, 16)])
                return carry2

            lax.fori_loop(0, CH, row_add, 0)
            pltpu.make_async_copy(
                bw.at[u], out_hbm.at[pl.ds((row0 + m) * CH, CH)],
                semw[u]).start()

            @pl.when(q < nquads - 1)
            def _():
                fire_gathers(u, m + 4)
        return carry

    lax.fori_loop(0, nquads, quad, 0)
    for k in range(4):
        drain_write(k)


_edgeout = functools.partial(
    pl.kernel,
    out_type=jax.ShapeDtypeStruct((E, F2), jnp.float32),
    mesh=plsc.VectorSubcoreMesh(core_axis_name="c", subcore_axis_name="s"),
    scratch_types=[
        pltpu.VMEM((2, 128, CH), jnp.int32),
        pltpu.VMEM((4, CH, F2), jnp.float32),
        pltpu.VMEM((4, CH, F2), jnp.float32),
        pltpu.VMEM((4, CH, F2), jnp.float32),
        pltpu.SemaphoreType.DMA, pltpu.SemaphoreType.DMA,
        pltpu.SemaphoreType.DMA, pltpu.SemaphoreType.DMA,
        pltpu.SemaphoreType.DMA, pltpu.SemaphoreType.DMA,
        pltpu.SemaphoreType.DMA, pltpu.SemaphoreType.DMA,
    ],
    compiler_params=pltpu.CompilerParams(use_tc_tiling_on_sc=False),
)(_edgeout_body)


def kernel(X, M, at_values, W, U, edge_time, edge_src, edge_trg):
    Minv = jnp.linalg.inv(M)

    et3 = edge_time.reshape(NB, ER_B, 128)
    es3 = edge_src.reshape(NB, ER_B, 128)
    etr3 = edge_trg.reshape(NB, ER_B, 128)

    P0, P1, SFT = _run_stage1(M, X, W, et3, es3, etr3)

    sft = jnp.pad(SFT.reshape(2, EROWS, 128),
                  ((0, 0), (0, ERP - EROWS), (0, 0)))
    a2 = jnp.pad(at_values.reshape(EROWS, 128), ((0, ERP - EROWS), (0, 0)))
    zeros = jnp.zeros((ZROWS, HALF), jnp.float32)

    S0, S1 = _segsum(P0.reshape(TN, HALF), P1.reshape(TN, HALF),
                     a2, sft, zeros)
    ZS, ZT = _run_proj(Minv, U, S0.reshape(T, NN, HALF),
                       S1.reshape(T, NN, HALF))
    out = _edgeout(ZS.reshape(TN, F2), ZT.reshape(TN, F2), sft)
    return out


# speedup vs baseline: 1.5176x; 1.5176x over previous
import functools

import jax
import jax.numpy as jnp
from jax import lax
from jax.experimental import pallas as pl
from jax.experimental.pallas import tpu as pltpu
from jax.experimental.pallas import tpu_sc as plsc

T = 8
NN = 10000
E = 512000
F0 = 128
F1 = 32
F2 = 32
HALF = 16
TN = T * NN

NB = 10
NBLK = NN // NB
EROWS = E // 128
ERP = 4096
ER_B = EROWS // NB

NC = 2
NS = 16
CH = 128

_PREC = lax.Precision.HIGHEST

_GDN = lax.GatherDimensionNumbers(
    offset_dims=(), collapsed_slice_dims=(0,), start_index_map=(0,))


def _bcast_lane(vec, lane):
    idx = jnp.full((16, 1), lane, jnp.int32)
    return lax.gather(vec, idx, _GDN, (1,),
                      mode=lax.GatherScatterMode.PROMISE_IN_BOUNDS)


def _stage1_body(m_ref, x_ref, w_ref, et_ref, es_ref, etr_ref,
                 p0_ref, p1_ref, sft_ref):
    w = w_ref[...]
    for t in range(T):
        xt = m_ref[t, 0] * x_ref[0]
        for u in range(1, T):
            xt = xt + m_ref[t, u] * x_ref[u]
        pt = lax.dot_general(
            xt, w[t], (((1,), (0,)), ((), ())),
            precision=_PREC, preferred_element_type=jnp.float32)
        p0_ref[t] = pt[:, :HALF]
        p1_ref[t] = pt[:, HALF:]
    tm = et_ref[...] * NN
    sft_ref[0] = tm + es_ref[...]
    sft_ref[1] = tm + etr_ref[...]


def _run_stage1(M, X, W, et3, es3, etr3):
    return pl.pallas_call(
        _stage1_body,
        grid=(NB,),
        in_specs=[
            pl.BlockSpec(memory_space=pltpu.SMEM),
            pl.BlockSpec((T, NBLK, F0), lambda i: (0, i, 0)),
            pl.BlockSpec((T, F0, F1), lambda i: (0, 0, 0)),
            pl.BlockSpec((1, ER_B, 128), lambda i: (i, 0, 0)),
            pl.BlockSpec((1, ER_B, 128), lambda i: (i, 0, 0)),
            pl.BlockSpec((1, ER_B, 128), lambda i: (i, 0, 0)),
        ],
        out_specs=[
            pl.BlockSpec((T, NBLK, HALF), lambda i: (0, i, 0)),
            pl.BlockSpec((T, NBLK, HALF), lambda i: (0, i, 0)),
            pl.BlockSpec((2, 1, ER_B, 128), lambda i: (0, i, 0, 0)),
        ],
        out_shape=[
            jax.ShapeDtypeStruct((T, NN, HALF), jnp.float32),
            jax.ShapeDtypeStruct((T, NN, HALF), jnp.float32),
            jax.ShapeDtypeStruct((2, NB, ER_B, 128), jnp.int32),
        ],
    )(M, X, W, et3, es3, etr3)


SB = 16
NSB = ERP // NS // SB
ZROWS = TN // NS


def _segsum_body(p0_hbm, p1_hbm, a_hbm, sft_hbm, z_hbm,
                 s0_hbm, s1_hbm,
                 acc, sft_blk, a_blk, rows, sbuf,
                 sg0, sg1, sg2, sg3, sa0, sa1, sa2, sa3):
    semg = (sg0, sg1, sg2, sg3)
    sema = (sa0, sa1, sa2, sa3)
    c = lax.axis_index("c")
    s = lax.axis_index("s")
    pltpu.sync_copy(z_hbm, acc.at[pl.ds(s * ZROWS, ZROWS)])
    plsc.subcore_barrier()

    base_row = s * (ERP // NS)

    def fire_gather(slot, k, r):
        @pl.when(c == 0)
        def _():
            pltpu.make_async_copy(
                p0_hbm.at[sft_blk.at[slot, 1, r]], rows.at[k], semg[k]).start()

        @pl.when(c == 1)
        def _():
            pltpu.make_async_copy(
                p1_hbm.at[sft_blk.at[slot, 1, r]], rows.at[k], semg[k]).start()

    def wait_gather(k):
        pltpu.make_async_copy(
            p0_hbm.at[pl.ds(0, CH)], rows.at[k], semg[k]).wait()

    def drain_scatter(k):
        pltpu.make_async_copy(
            p0_hbm.at[pl.ds(0, CH)], sbuf.at[k], sema[k]).wait()

    def scale(slot, k, r):
        def grp(g, carry):
            a_vec = a_blk[slot, r, pl.ds(g * 16, 16)]
            base = g * 16
            for ee in range(16):
                bc = _bcast_lane(a_vec, ee)
                sbuf[k, base + ee] = rows[k, base + ee] * bc
            return carry

        lax.fori_loop(0, CH // 16, grp, 0)

    def fire_scatter(slot, k, r):
        pltpu.make_async_copy(
            sbuf.at[k], acc.at[sft_blk.at[slot, 0, r]], sema[k]).start(add=True)

    def run_superblock(sb_idx, slot, first_pred):
        r0 = base_row + sb_idx * SB
        pltpu.sync_copy(sft_hbm.at[:, pl.ds(r0, SB)], sft_blk.at[slot])
        pltpu.sync_copy(a_hbm.at[pl.ds(r0, SB)], a_blk.at[slot])
        for k in range(4):
            fire_gather(slot, k, k)

        def chunk(u, r, drain):
            wait_gather(u)
            if drain == "always":
                drain_scatter(u)
            elif drain == "cond":
                @pl.when(jnp.logical_not(first_pred))
                def _():
                    drain_scatter(u)
            scale(slot, u, r)
            fire_scatter(slot, u, r)

        for u in range(4):
            chunk(u, u, "cond" if first_pred is not None else "always")
            fire_gather(slot, u, u + 4)

        def quad(q, carry):
            for u in range(4):
                r = q * 4 + u
                chunk(u, r, "always")
                fire_gather(slot, u, r + 4)
            return carry

        lax.fori_loop(1, SB // 4 - 1, quad, 0)
        for u in range(4):
            chunk(u, SB - 4 + u, "always")

    def sbpair(p, carry):
        run_superblock(p * 2, 0, p == 0)
        run_superblock(p * 2 + 1, 1, None)
        return carry

    lax.fori_loop(0, NSB // 2, sbpair, 0)
    for k in range(4):
        drain_scatter(k)
    plsc.subcore_barrier()

    wr0 = s * ZROWS

    @pl.when(c == 0)
    def _():
        pltpu.sync_copy(acc.at[pl.ds(wr0, ZROWS)], s0_hbm.at[pl.ds(wr0, ZROWS)])

    @pl.when(c == 1)
    def _():
        pltpu.sync_copy(acc.at[pl.ds(wr0, ZROWS)], s1_hbm.at[pl.ds(wr0, ZROWS)])


_segsum = functools.partial(
    pl.kernel,
    out_type=[jax.ShapeDtypeStruct((TN, HALF), jnp.float32),
              jax.ShapeDtypeStruct((TN, HALF), jnp.float32)],
    mesh=plsc.VectorSubcoreMesh(core_axis_name="c", subcore_axis_name="s"),
    scratch_types=[
        pltpu.VMEM_SHARED((TN, HALF), jnp.float32),
        pltpu.VMEM((2, 2, SB, CH), jnp.int32),
        pltpu.VMEM((2, SB, CH), jnp.float32),
        pltpu.VMEM((4, CH, HALF), jnp.float32),
        pltpu.VMEM((4, CH, HALF), jnp.float32),
        pltpu.SemaphoreType.DMA, pltpu.SemaphoreType.DMA,
        pltpu.SemaphoreType.DMA, pltpu.SemaphoreType.DMA,
        pltpu.SemaphoreType.DMA, pltpu.SemaphoreType.DMA,
        pltpu.SemaphoreType.DMA, pltpu.SemaphoreType.DMA,
    ],
    compiler_params=pltpu.CompilerParams(use_tc_tiling_on_sc=False),
)(_segsum_body)


def _proj_body(minv_ref, u_ref, s0_ref, s1_ref, zs_ref, zt_ref):
    u = u_ref[...]
    u0s, u1s = u[0:HALF], u[HALF:2 * HALF]
    u0t, u1t = u[2 * HALF:3 * HALF], u[3 * HALF:]
    for t in range(T):
        y0 = minv_ref[t, 0] * s0_ref[0]
        y1 = minv_ref[t, 0] * s1_ref[0]
        for uu in range(1, T):
            y0 = y0 + minv_ref[t, uu] * s0_ref[uu]
            y1 = y1 + minv_ref[t, uu] * s1_ref[uu]
        zs_ref[t] = (
            lax.dot_general(y0, u0s, (((1,), (0,)), ((), ())),
                            precision=_PREC, preferred_element_type=jnp.float32)
            + lax.dot_general(y1, u1s, (((1,), (0,)), ((), ())),
                              precision=_PREC, preferred_element_type=jnp.float32))
        zt_ref[t] = (
            lax.dot_general(y0, u0t, (((1,), (0,)), ((), ())),
                            precision=_PREC, preferred_element_type=jnp.float32)
            + lax.dot_general(y1, u1t, (((1,), (0,)), ((), ())),
                              precision=_PREC, preferred_element_type=jnp.float32))


def _run_proj(Minv, U, S0, S1):
    return pl.pallas_call(
        _proj_body,
        grid=(NB,),
        in_specs=[
            pl.BlockSpec(memory_space=pltpu.SMEM),
            pl.BlockSpec((2 * F1, F2), lambda i: (0, 0)),
            pl.BlockSpec((T, NBLK, HALF), lambda i: (0, i, 0)),
            pl.BlockSpec((T, NBLK, HALF), lambda i: (0, i, 0)),
        ],
        out_specs=[
            pl.BlockSpec((T, NBLK, F2), lambda i: (0, i, 0)),
            pl.BlockSpec((T, NBLK, F2), lambda i: (0, i, 0)),
        ],
        out_shape=[
            jax.ShapeDtypeStruct((T, NN, F2), jnp.float32),
            jax.ShapeDtypeStruct((T, NN, F2), jnp.float32),
        ],
    )(Minv, U, S0, S1)


NW = NC * NS
PAGES = EROWS // 8


def _edgeout_body(zs_hbm, zt_hbm, sft_hbm, out_hbm,
                  sftw, bs, bt, bw,
                  sg0, sg1, sg2, sg3, sw0, sw1, sw2, sw3):
    semg = (sg0, sg1, sg2, sg3)
    semw = (sw0, sw1, sw2, sw3)
    c = lax.axis_index("c")
    s = lax.axis_index("s")
    wid = s * NC + c
    extra = PAGES % NW
    npages = jnp.int32(PAGES // NW) + (wid < extra).astype(jnp.int32)
    page0 = jnp.where(wid < extra, wid * (PAGES // NW + 1),
                      extra + wid * (PAGES // NW))
    row0 = page0 * 8
    nquads = npages * 2

    pltpu.sync_copy(sft_hbm.at[:, pl.ds(row0, 128)], sftw)

    def fire_gathers(k, m):
        pltpu.make_async_copy(
            zs_hbm.at[sftw.at[0, m]], bs.at[k], semg[k]).start()
        pltpu.make_async_copy(
            zt_hbm.at[sftw.at[1, m]], bt.at[k], semg[k]).start()

    def wait_gathers(k):
        pltpu.make_async_copy(
            zs_hbm.at[pl.ds(0, CH)], bs.at[k], semg[k]).wait()
        pltpu.make_async_copy(
            zs_hbm.at[pl.ds(0, CH)], bt.at[k], semg[k]).wait()

    def drain_write(k):
        pltpu.make_async_copy(
            zs_hbm.at[pl.ds(0, CH)], bw.at[k], semw[k]).wait()

    for k in range(4):
        fire_gathers(k, k)

    def quad(q, carry):
        for u in range(4):
            m = q * 4 + u
            wait_gathers(u)

            @pl.when(q > 0)
            def _():
                drain_write(u)

            def row_add(r, carry2):
                bw[u, r, pl.ds(0, 16)] = (bs[u, r, pl.ds(0, 16)]
                                          + bt[u, r, pl.ds(0, 16)])
                bw[u, r, pl.ds(16, 16)] = (bs[u, r, pl.ds(16, 16)]
                                           + bt[u, r, pl.ds(16, 16)])
                return carry2

            lax.fori_loop(0, CH, row_add, 0)
            pltpu.make_async_copy(
                bw.at[u], out_hbm.at[pl.ds((row0 + m) * CH, CH)],
                semw[u]).start()

            @pl.when(q < nquads - 1)
            def _():
                fire_gathers(u, m + 4)
        return carry

    lax.fori_loop(0, nquads, quad, 0)
    for k in range(4):
        drain_write(k)


_edgeout = functools.partial(
    pl.kernel,
    out_type=jax.ShapeDtypeStruct((E, F2), jnp.float32),
    mesh=plsc.VectorSubcoreMesh(core_axis_name="c", subcore_axis_name="s"),
    scratch_types=[
        pltpu.VMEM((2, 128, CH), jnp.int32),
        pltpu.VMEM((4, CH, F2), jnp.float32),
        pltpu.VMEM((4, CH, F2), jnp.float32),
        pltpu.VMEM((4, CH, F2), jnp.float32),
        pltpu.SemaphoreType.DMA, pltpu.SemaphoreType.DMA,
        pltpu.SemaphoreType.DMA, pltpu.SemaphoreType.DMA,
        pltpu.SemaphoreType.DMA, pltpu.SemaphoreType.DMA,
        pltpu.SemaphoreType.DMA, pltpu.SemaphoreType.DMA,
    ],
    compiler_params=pltpu.CompilerParams(use_tc_tiling_on_sc=False),
)(_edgeout_body)


def kernel(X, M, at_values, W, U, edge_time, edge_src, edge_trg):
    Minv = jnp.linalg.inv(M)

    et3 = edge_time.reshape(NB, ER_B, 128)
    es3 = edge_src.reshape(NB, ER_B, 128)
    etr3 = edge_trg.reshape(NB, ER_B, 128)

    P0, P1, SFT = _run_stage1(M, X, W, et3, es3, etr3)

    sft = jnp.pad(SFT.reshape(2, EROWS, 128),
                  ((0, 0), (0, ERP - EROWS), (0, 0)))
    a2 = jnp.pad(at_values.reshape(EROWS, 128), ((0, ERP - EROWS), (0, 0)))
    zeros = jnp.zeros((ZROWS, HALF), jnp.float32)

    S0, S1 = _segsum(P0.reshape(TN, HALF), P1.reshape(TN, HALF),
                     a2, sft, zeros)
    ZS, ZT = _run_proj(Minv, U, S0.reshape(T, NN, HALF),
                       S1.reshape(T, NN, HALF))
    out = _edgeout(ZS.reshape(TN, F2), ZT.reshape(TN, F2), sft)
    return out


# speedup vs baseline: 1.5511x; 1.0221x over previous
import functools

import jax
import jax.numpy as jnp
from jax import lax
from jax.experimental import pallas as pl
from jax.experimental.pallas import tpu as pltpu
from jax.experimental.pallas import tpu_sc as plsc

T = 8
NN = 10000
E = 512000
F0 = 128
F1 = 32
F2 = 32
HALF = 16
TN = T * NN

NB = 10
NBLK = NN // NB
EROWS = E // 128
ERP = 4096
ER_B = EROWS // NB

NC = 2
NS = 16
CH = 128

_PREC = lax.Precision.HIGHEST

_GDN = lax.GatherDimensionNumbers(
    offset_dims=(), collapsed_slice_dims=(0,), start_index_map=(0,))


def _bcast_lane(vec, lane):
    idx = jnp.full((16, 1), lane, jnp.int32)
    return lax.gather(vec, idx, _GDN, (1,),
                      mode=lax.GatherScatterMode.PROMISE_IN_BOUNDS)


def _stage1_body(m_ref, x_ref, w_ref, et_ref, es_ref, etr_ref,
                 p0_ref, p1_ref, sft_ref):
    w = w_ref[...]
    for t in range(T):
        xt = m_ref[t, 0] * x_ref[0]
        for u in range(1, T):
            xt = xt + m_ref[t, u] * x_ref[u]
        pt = lax.dot_general(
            xt, w[t], (((1,), (0,)), ((), ())),
            precision=_PREC, preferred_element_type=jnp.float32)
        p0_ref[t] = pt[:, :HALF]
        p1_ref[t] = pt[:, HALF:]
    tm = et_ref[...] * NN
    sft_ref[0] = tm + es_ref[...]
    sft_ref[1] = tm + etr_ref[...]


def _run_stage1(M, X, W, et3, es3, etr3):
    return pl.pallas_call(
        _stage1_body,
        grid=(NB,),
        in_specs=[
            pl.BlockSpec(memory_space=pltpu.SMEM),
            pl.BlockSpec((T, NBLK, F0), lambda i: (0, i, 0)),
            pl.BlockSpec((T, F0, F1), lambda i: (0, 0, 0)),
            pl.BlockSpec((1, ER_B, 128), lambda i: (i, 0, 0)),
            pl.BlockSpec((1, ER_B, 128), lambda i: (i, 0, 0)),
            pl.BlockSpec((1, ER_B, 128), lambda i: (i, 0, 0)),
        ],
        out_specs=[
            pl.BlockSpec((T, NBLK, HALF), lambda i: (0, i, 0)),
            pl.BlockSpec((T, NBLK, HALF), lambda i: (0, i, 0)),
            pl.BlockSpec((2, 1, ER_B, 128), lambda i: (0, i, 0, 0)),
        ],
        out_shape=[
            jax.ShapeDtypeStruct((T, NN, HALF), jnp.float32),
            jax.ShapeDtypeStruct((T, NN, HALF), jnp.float32),
            jax.ShapeDtypeStruct((2, NB, ER_B, 128), jnp.int32),
        ],
    )(M, X, W, et3, es3, etr3)


SB = 16
NSB = ERP // NS // SB
ZROWS = TN // NS


def _segsum_body(p0_hbm, p1_hbm, a_hbm, sft_hbm, z_hbm,
                 s0_hbm, s1_hbm,
                 acc, sft_blk, a_blk, rows, sbuf,
                 sg0, sg1, sg2, sg3, sa0, sa1, sa2, sa3, si0, si1):
    semg = (sg0, sg1, sg2, sg3)
    sema = (sa0, sa1, sa2, sa3)
    semi = (si0, si1)
    c = lax.axis_index("c")
    s = lax.axis_index("s")
    pltpu.sync_copy(z_hbm, acc.at[pl.ds(s * ZROWS, ZROWS)])
    plsc.subcore_barrier()

    base_row = s * (ERP // NS)

    def fire_gather(slot, k, r):
        @pl.when(c == 0)
        def _():
            pltpu.make_async_copy(
                p0_hbm.at[sft_blk.at[slot, 1, r]], rows.at[k], semg[k]).start()

        @pl.when(c == 1)
        def _():
            pltpu.make_async_copy(
                p1_hbm.at[sft_blk.at[slot, 1, r]], rows.at[k], semg[k]).start()

    def wait_gather(k):
        pltpu.make_async_copy(
            p0_hbm.at[pl.ds(0, CH)], rows.at[k], semg[k]).wait()

    def drain_scatter(k):
        pltpu.make_async_copy(
            p0_hbm.at[pl.ds(0, CH)], sbuf.at[k], sema[k]).wait()

    def scale(slot, k, r):
        def grp(g, carry):
            a_vec = a_blk[slot, r, pl.ds(g * 16, 16)]
            base = g * 16
            for ee in range(16):
                bc = _bcast_lane(a_vec, ee)
                sbuf[k, base + ee] = rows[k, base + ee] * bc
            return carry

        lax.fori_loop(0, CH // 16, grp, 0)

    def fire_scatter(slot, k, r):
        pltpu.make_async_copy(
            sbuf.at[k], acc.at[sft_blk.at[slot, 0, r]], sema[k]).start(add=True)

    def fire_idx(sb_idx, slot):
        r0 = base_row + sb_idx * SB
        pltpu.make_async_copy(
            sft_hbm.at[:, pl.ds(r0, SB)], sft_blk.at[slot], semi[slot]).start()
        pltpu.make_async_copy(
            a_hbm.at[pl.ds(r0, SB)], a_blk.at[slot], semi[slot]).start()

    def run_superblock(sb_idx, slot, first_pred, prefetch_pred):
        pltpu.make_async_copy(
            sft_hbm.at[:, pl.ds(0, SB)], sft_blk.at[slot], semi[slot]).wait()
        pltpu.make_async_copy(
            a_hbm.at[pl.ds(0, SB)], a_blk.at[slot], semi[slot]).wait()
        if prefetch_pred is None:
            fire_idx(sb_idx + 1, 1 - slot)
        else:
            @pl.when(prefetch_pred)
            def _():
                fire_idx(sb_idx + 1, 1 - slot)
        for k in range(4):
            fire_gather(slot, k, k)

        def chunk(u, r, drain):
            wait_gather(u)
            if drain == "always":
                drain_scatter(u)
            elif drain == "cond":
                @pl.when(jnp.logical_not(first_pred))
                def _():
                    drain_scatter(u)
            scale(slot, u, r)
            fire_scatter(slot, u, r)

        for u in range(4):
            chunk(u, u, "cond" if first_pred is not None else "always")
            fire_gather(slot, u, u + 4)

        def quad(q, carry):
            for u in range(4):
                r = q * 4 + u
                chunk(u, r, "always")
                fire_gather(slot, u, r + 4)
            return carry

        lax.fori_loop(1, SB // 4 - 1, quad, 0)
        for u in range(4):
            chunk(u, SB - 4 + u, "always")

    fire_idx(0, 0)

    def sbpair(p, carry):
        run_superblock(p * 2, 0, p == 0, None)
        run_superblock(p * 2 + 1, 1, None, p < NSB // 2 - 1)
        return carry

    lax.fori_loop(0, NSB // 2, sbpair, 0)
    for k in range(4):
        drain_scatter(k)
    plsc.subcore_barrier()

    wr0 = s * ZROWS

    @pl.when(c == 0)
    def _():
        pltpu.sync_copy(acc.at[pl.ds(wr0, ZROWS)], s0_hbm.at[pl.ds(wr0, ZROWS)])

    @pl.when(c == 1)
    def _():
        pltpu.sync_copy(acc.at[pl.ds(wr0, ZROWS)], s1_hbm.at[pl.ds(wr0, ZROWS)])


_segsum = functools.partial(
    pl.kernel,
    out_type=[jax.ShapeDtypeStruct((TN, HALF), jnp.float32),
              jax.ShapeDtypeStruct((TN, HALF), jnp.float32)],
    mesh=plsc.VectorSubcoreMesh(core_axis_name="c", subcore_axis_name="s"),
    scratch_types=[
        pltpu.VMEM_SHARED((TN, HALF), jnp.float32),
        pltpu.VMEM((2, 2, SB, CH), jnp.int32),
        pltpu.VMEM((2, SB, CH), jnp.float32),
        pltpu.VMEM((4, CH, HALF), jnp.float32),
        pltpu.VMEM((4, CH, HALF), jnp.float32),
        pltpu.SemaphoreType.DMA, pltpu.SemaphoreType.DMA,
        pltpu.SemaphoreType.DMA, pltpu.SemaphoreType.DMA,
        pltpu.SemaphoreType.DMA, pltpu.SemaphoreType.DMA,
        pltpu.SemaphoreType.DMA, pltpu.SemaphoreType.DMA,
        pltpu.SemaphoreType.DMA, pltpu.SemaphoreType.DMA,
    ],
    compiler_params=pltpu.CompilerParams(use_tc_tiling_on_sc=False),
)(_segsum_body)


def _proj_body(minv_ref, u_ref, s0_ref, s1_ref, zs_ref, zt_ref):
    u = u_ref[...]
    u0s, u1s = u[0:HALF], u[HALF:2 * HALF]
    u0t, u1t = u[2 * HALF:3 * HALF], u[3 * HALF:]
    for t in range(T):
        y0 = minv_ref[t, 0] * s0_ref[0]
        y1 = minv_ref[t, 0] * s1_ref[0]
        for uu in range(1, T):
            y0 = y0 + minv_ref[t, uu] * s0_ref[uu]
            y1 = y1 + minv_ref[t, uu] * s1_ref[uu]
        zs_ref[t] = (
            lax.dot_general(y0, u0s, (((1,), (0,)), ((), ())),
                            precision=_PREC, preferred_element_type=jnp.float32)
            + lax.dot_general(y1, u1s, (((1,), (0,)), ((), ())),
                              precision=_PREC, preferred_element_type=jnp.float32))
        zt_ref[t] = (
            lax.dot_general(y0, u0t, (((1,), (0,)), ((), ())),
                            precision=_PREC, preferred_element_type=jnp.float32)
            + lax.dot_general(y1, u1t, (((1,), (0,)), ((), ())),
                              precision=_PREC, preferred_element_type=jnp.float32))


def _run_proj(Minv, U, S0, S1):
    return pl.pallas_call(
        _proj_body,
        grid=(NB,),
        in_specs=[
            pl.BlockSpec(memory_space=pltpu.SMEM),
            pl.BlockSpec((2 * F1, F2), lambda i: (0, 0)),
            pl.BlockSpec((T, NBLK, HALF), lambda i: (0, i, 0)),
            pl.BlockSpec((T, NBLK, HALF), lambda i: (0, i, 0)),
        ],
        out_specs=[
            pl.BlockSpec((T, NBLK, F2), lambda i: (0, i, 0)),
            pl.BlockSpec((T, NBLK, F2), lambda i: (0, i, 0)),
        ],
        out_shape=[
            jax.ShapeDtypeStruct((T, NN, F2), jnp.float32),
            jax.ShapeDtypeStruct((T, NN, F2), jnp.float32),
        ],
    )(Minv, U, S0, S1)


NW = NC * NS
PAGES = EROWS // 8


def _edgeout_body(zs_hbm, zt_hbm, sft_hbm, out_hbm,
                  sftw, bs, bt, bw,
                  sg0, sg1, sg2, sg3, sw0, sw1, sw2, sw3):
    semg = (sg0, sg1, sg2, sg3)
    semw = (sw0, sw1, sw2, sw3)
    c = lax.axis_index("c")
    s = lax.axis_index("s")
    wid = s * NC + c
    extra = PAGES % NW
    npages = jnp.int32(PAGES // NW) + (wid < extra).astype(jnp.int32)
    page0 = jnp.where(wid < extra, wid * (PAGES // NW + 1),
                      extra + wid * (PAGES // NW))
    row0 = page0 * 8
    nquads = npages * 2

    pltpu.sync_copy(sft_hbm.at[:, pl.ds(row0, 128)], sftw)

    def fire_gathers(k, m):
        pltpu.make_async_copy(
            zs_hbm.at[sftw.at[0, m]], bs.at[k], semg[k]).start()
        pltpu.make_async_copy(
            zt_hbm.at[sftw.at[1, m]], bt.at[k], semg[k]).start()

    def wait_gathers(k):
        pltpu.make_async_copy(
            zs_hbm.at[pl.ds(0, CH)], bs.at[k], semg[k]).wait()
        pltpu.make_async_copy(
            zs_hbm.at[pl.ds(0, CH)], bt.at[k], semg[k]).wait()

    def drain_write(k):
        pltpu.make_async_copy(
            zs_hbm.at[pl.ds(0, CH)], bw.at[k], semw[k]).wait()

    for k in range(4):
        fire_gathers(k, k)

    def quad(q, carry):
        for u in range(4):
            m = q * 4 + u
            wait_gathers(u)

            @pl.when(q > 0)
            def _():
                drain_write(u)

            def row_add(r, carry2):
                bw[u, r, pl.ds(0, 16)] = (bs[u, r, pl.ds(0, 16)]
                                          + bt[u, r, pl.ds(0, 16)])
                bw[u, r, pl.ds(16, 16)] = (bs[u, r, pl.ds(16, 16)]
                                           + bt[u, r, pl.ds(16, 16)])
                return carry2

            lax.fori_loop(0, CH, row_add, 0)
            pltpu.make_async_copy(
                bw.at[u], out_hbm.at[pl.ds((row0 + m) * CH, CH)],
                semw[u]).start()

            @pl.when(q < nquads - 1)
            def _():
                fire_gathers(u, m + 4)
        return carry

    lax.fori_loop(0, nquads, quad, 0)
    for k in range(4):
        drain_write(k)


_edgeout = functools.partial(
    pl.kernel,
    out_type=jax.ShapeDtypeStruct((E, F2), jnp.float32),
    mesh=plsc.VectorSubcoreMesh(core_axis_name="c", subcore_axis_name="s"),
    scratch_types=[
        pltpu.VMEM((2, 128, CH), jnp.int32),
        pltpu.VMEM((4, CH, F2), jnp.float32),
        pltpu.VMEM((4, CH, F2), jnp.float32),
        pltpu.VMEM((4, CH, F2), jnp.float32),
        pltpu.SemaphoreType.DMA, pltpu.SemaphoreType.DMA,
        pltpu.SemaphoreType.DMA, pltpu.SemaphoreType.DMA,
        pltpu.SemaphoreType.DMA, pltpu.SemaphoreType.DMA,
        pltpu.SemaphoreType.DMA, pltpu.SemaphoreType.DMA,
    ],
    compiler_params=pltpu.CompilerParams(use_tc_tiling_on_sc=False),
)(_edgeout_body)


def kernel(X, M, at_values, W, U, edge_time, edge_src, edge_trg):
    Minv = jnp.linalg.inv(M)

    et3 = edge_time.reshape(NB, ER_B, 128)
    es3 = edge_src.reshape(NB, ER_B, 128)
    etr3 = edge_trg.reshape(NB, ER_B, 128)

    P0, P1, SFT = _run_stage1(M, X, W, et3, es3, etr3)

    sft = jnp.pad(SFT.reshape(2, EROWS, 128),
                  ((0, 0), (0, ERP - EROWS), (0, 0)))
    a2 = jnp.pad(at_values.reshape(EROWS, 128), ((0, ERP - EROWS), (0, 0)))
    zeros = jnp.zeros((ZROWS, HALF), jnp.float32)

    S0, S1 = _segsum(P0.reshape(TN, HALF), P1.reshape(TN, HALF),
                     a2, sft, zeros)
    ZS, ZT = _run_proj(Minv, U, S0.reshape(T, NN, HALF),
                       S1.reshape(T, NN, HALF))
    out = _edgeout(ZS.reshape(TN, F2), ZT.reshape(TN, F2), sft)
    return out
